# baseline (speedup 1.0000x reference)
import functools

import jax
import jax.numpy as jnp
from jax import lax
from jax.experimental import pallas as pl
from jax.experimental.pallas import tpu as pltpu
from jax.experimental.pallas import tpu_sc as plsc

_TOPK = 8
_LANES = 16
_NCORES = 2
_NSUB = 16
_NW = _NCORES * _NSUB


def _proj_body(x_ref, wt_ref, bias_ref, st_out_ref):
    x = x_ref[...]
    wt = wt_ref[...]
    scores = jnp.dot(x, wt, preferred_element_type=jnp.float32)
    scores = scores + bias_ref[...]
    st_out_ref[...] = scores.T


@functools.partial(jax.jit, static_argnames=("block_rows",))
def _proj_t(x, weight, bias, block_rows=1024):
    n_rows, k = x.shape
    n_experts = weight.shape[0]
    wt = weight.T
    bias2d = bias.reshape(1, n_experts)
    grid = (n_rows // block_rows,)
    return pl.pallas_call(
        _proj_body,
        grid=grid,
        in_specs=[
            pl.BlockSpec((block_rows, k), lambda i: (i, 0)),
            pl.BlockSpec((k, n_experts), lambda i: (0, 0)),
            pl.BlockSpec((1, n_experts), lambda i: (0, 0)),
        ],
        out_specs=pl.BlockSpec((n_experts, block_rows), lambda i: (0, i)),
        out_shape=jax.ShapeDtypeStruct((n_experts, n_rows), jnp.float32),
        compiler_params=pltpu.CompilerParams(
            dimension_semantics=("arbitrary",),
        ),
    )(x, wt, bias2d)


def _make_topk_sc(n_rows, n_experts):
    rows_per = n_rows // _NW
    n_groups = rows_per // _LANES
    mesh = plsc.VectorSubcoreMesh(core_axis_name="c", subcore_axis_name="s")

    @functools.partial(
        pl.kernel,
        mesh=mesh,
        out_type=[
            jax.ShapeDtypeStruct((_TOPK, n_rows), jnp.float32),
            jax.ShapeDtypeStruct((_TOPK, n_rows), jnp.int32),
        ],
        scratch_types=[
            pltpu.VMEM((n_experts, rows_per), jnp.float32),
            pltpu.VMEM((_TOPK, rows_per), jnp.float32),
            pltpu.VMEM((_TOPK, rows_per), jnp.int32),
        ],
    )
    def topk_sc(st_hbm, w_out, i_out, st_v, w_v, i_v):
        wid = lax.axis_index("s") * _NCORES + lax.axis_index("c")
        base = wid * rows_per
        pltpu.sync_copy(st_hbm.at[:, pl.ds(base, rows_per)], st_v)

        def group_body(g, _):
            col = g * _LANES
            tv = [jnp.full((_LANES,), -jnp.inf, jnp.float32)
                  for _ in range(_TOPK)]
            ti = [jnp.zeros((_LANES,), jnp.int32) for _ in range(_TOPK)]
            for e in range(n_experts):
                c = st_v[e, pl.ds(col, _LANES)]
                ci = jnp.full((_LANES,), e, jnp.int32)
                for j in range(_TOPK):
                    m = c > tv[j]
                    tv_new = jnp.where(m, c, tv[j])
                    c = jnp.where(m, tv[j], c)
                    ti_new = jnp.where(m, ci, ti[j])
                    ci = jnp.where(m, ti[j], ci)
                    tv[j] = tv_new
                    ti[j] = ti_new
            es = [jnp.exp(tv[j] - tv[0]) for j in range(_TOPK)]
            tot = es[0]
            for j in range(1, _TOPK):
                tot = tot + es[j]
            for j in range(_TOPK):
                w_v[j, pl.ds(col, _LANES)] = es[j] / tot
                i_v[j, pl.ds(col, _LANES)] = ti[j]
            return _

        lax.fori_loop(0, n_groups, group_body, None)
        pltpu.sync_copy(w_v, w_out.at[:, pl.ds(base, rows_per)])
        pltpu.sync_copy(i_v, i_out.at[:, pl.ds(base, rows_per)])

    return topk_sc


@jax.jit
def _moe_gate(x, weight, bias):
    n_rows = x.shape[0]
    n_experts = weight.shape[0]
    st = _proj_t(x, weight, bias)
    w_t, i_t = _make_topk_sc(n_rows, n_experts)(st)
    return w_t.T, i_t.T


def kernel(x, weight, bias):
    w, i = _moe_gate(x, weight, bias)
    return w.astype(x.dtype), i


# speedup vs baseline: 1.3426x; 1.3426x over previous
import functools

import jax
import jax.numpy as jnp
from jax.experimental import pallas as pl
from jax.experimental.pallas import tpu as pltpu

_TOPK = 8


def _gate_body(x_ref, wt_ref, bias_ref, w_out_ref, i_out_ref, *, n_experts):
    x = x_ref[...]
    wt = wt_ref[...]
    scores = jnp.dot(x, wt, preferred_element_type=jnp.float32)
    scores = scores + bias_ref[...]

    st = scores.T
    idx = jax.lax.broadcasted_iota(jnp.int32, st.shape, 0).astype(jnp.float32)
    vals = st
    top_vs = []
    top_is = []
    for _ in range(_TOPK):
        m = jnp.max(vals, axis=0, keepdims=True)
        am = jnp.min(jnp.where(vals == m, idx, float(n_experts)), axis=0,
                     keepdims=True)
        top_vs.append(m)
        top_is.append(am)
        vals = jnp.where(idx == am, -jnp.inf, vals)

    tv = jnp.concatenate(top_vs, axis=0)
    ti = jnp.concatenate(top_is, axis=0)
    e = jnp.exp(tv - tv[0:1])
    w = e / jnp.sum(e, axis=0, keepdims=True)
    w_out_ref[...] = w.T
    i_out_ref[...] = ti.T.astype(jnp.int32)


@functools.partial(jax.jit, static_argnames=("block_rows",))
def _moe_gate(x, weight, bias, block_rows=1024):
    n_rows, k = x.shape
    n_experts = weight.shape[0]
    wt = weight.T
    bias2d = bias.reshape(1, n_experts)

    grid = (n_rows // block_rows,)
    out_w, out_i = pl.pallas_call(
        functools.partial(_gate_body, n_experts=n_experts),
        grid=grid,
        in_specs=[
            pl.BlockSpec((block_rows, k), lambda i: (i, 0)),
            pl.BlockSpec((k, n_experts), lambda i: (0, 0)),
            pl.BlockSpec((1, n_experts), lambda i: (0, 0)),
        ],
        out_specs=[
            pl.BlockSpec((block_rows, _TOPK), lambda i: (i, 0)),
            pl.BlockSpec((block_rows, _TOPK), lambda i: (i, 0)),
        ],
        out_shape=[
            jax.ShapeDtypeStruct((n_rows, _TOPK), jnp.float32),
            jax.ShapeDtypeStruct((n_rows, _TOPK), jnp.int32),
        ],
        compiler_params=pltpu.CompilerParams(
            dimension_semantics=("arbitrary",),
        ),
    )(x, wt, bias2d)
    return out_w, out_i


def kernel(x, weight, bias):
    w, i = _moe_gate(x, weight, bias)
    return w.astype(x.dtype), i


# speedup vs baseline: 1.5468x; 1.1521x over previous
import functools

import jax
import jax.numpy as jnp
from jax.experimental import pallas as pl
from jax.experimental.pallas import tpu as pltpu

_TOPK = 8


def _gate_body(xa_ref, xb_ref, wta_ref, wtb_ref, bias_ref, w_out_ref,
               i_out_ref, *, n_experts):
    scores = (jnp.dot(xa_ref[...], wta_ref[...],
                      preferred_element_type=jnp.float32)
              + jnp.dot(xb_ref[...], wtb_ref[...],
                        preferred_element_type=jnp.float32))
    scores = scores + bias_ref[...]

    st = scores.T
    idx = jax.lax.broadcasted_iota(jnp.int32, st.shape, 0).astype(jnp.float32)
    vals = st
    top_vs = []
    top_is = []
    for _ in range(_TOPK):
        m = jnp.max(vals, axis=0, keepdims=True)
        am = jnp.min(jnp.where(vals == m, idx, float(n_experts)), axis=0,
                     keepdims=True)
        top_vs.append(m)
        top_is.append(am)
        vals = jnp.where(idx == am, -jnp.inf, vals)

    tv = jnp.concatenate(top_vs, axis=0)
    ti = jnp.concatenate(top_is, axis=0)
    e = jnp.exp(tv - tv[0:1])
    w = e / jnp.sum(e, axis=0, keepdims=True)
    w_out_ref[...] = w
    i_out_ref[...] = ti.astype(jnp.int32)


@functools.partial(jax.jit, static_argnames=("block_rows",))
def _moe_gate(x, weight, bias, block_rows=1024):
    n_rows, k = x.shape
    n_experts = weight.shape[0]
    wt = weight.T
    bias2d = bias.reshape(1, n_experts)

    grid = (n_rows // block_rows,)
    out_w, out_i = pl.pallas_call(
        functools.partial(_gate_body, n_experts=n_experts),
        grid=grid,
        in_specs=[
            pl.BlockSpec((block_rows, k // 2), lambda i: (i, 0)),
            pl.BlockSpec((block_rows, k // 2), lambda i: (i, 1)),
            pl.BlockSpec((k // 2, n_experts), lambda i: (0, 0)),
            pl.BlockSpec((k // 2, n_experts), lambda i: (1, 0)),
            pl.BlockSpec((1, n_experts), lambda i: (0, 0)),
        ],
        out_specs=[
            pl.BlockSpec((_TOPK, block_rows), lambda i: (0, i)),
            pl.BlockSpec((_TOPK, block_rows), lambda i: (0, i)),
        ],
        out_shape=[
            jax.ShapeDtypeStruct((_TOPK, n_rows), jnp.float32),
            jax.ShapeDtypeStruct((_TOPK, n_rows), jnp.int32),
        ],
        compiler_params=pltpu.CompilerParams(
            dimension_semantics=("arbitrary",),
        ),
    )(x, x, wt, wt, bias2d)
    return out_w.T, out_i.T


def kernel(x, weight, bias):
    w, i = _moe_gate(x, weight, bias)
    return w.astype(x.dtype), i
